# Initial kernel scaffold; baseline (speedup 1.0000x reference)
import jax
import jax.numpy as jnp
from jax.experimental import pallas as pl


def kernel(warped_events, pol_mask, ts_list, tref, ts_scaling):
    raise NotImplementedError("write your pallas kernel here")


# speedup vs baseline: 3.6750x; 3.6750x over previous
import functools
import jax
import jax.numpy as jnp
from jax import lax
from jax.experimental import pallas as pl
from jax.experimental.pallas import tpu as pltpu
from jax.experimental.pallas import tpu_sc as plsc

H, W = 480, 640
HW = H * W
PLANE = 2 * HW
NC, NS = 2, 16
CH = 1952
NCHUNK = 8
PER_TILE = CH * NCHUNK
STRIPE = PLANE // NS


def _make_sc_kernel(B, N):
    assert B % NC == 0
    BPC = B // NC
    covered = NS * PER_TILE
    tail = N - covered
    assert 0 <= tail <= CH and tail % 16 == 0 and covered % 8 == 0
    TGROUPS = tail // 16
    G = CH // 16

    mesh = plsc.VectorSubcoreMesh(core_axis_name="c", subcore_axis_name="s")

    @functools.partial(
        pl.kernel,
        out_type=[
            jax.ShapeDtypeStruct((B * PLANE,), jnp.float32),
            jax.ShapeDtypeStruct((B * PLANE,), jnp.float32),
        ],
        mesh=mesh,
        compiler_params=pltpu.CompilerParams(needs_layout_passes=False),
        scratch_types=[
            pltpu.VMEM((2 * CH,), jnp.float32),
            pltpu.VMEM((2 * CH,), jnp.float32),
            pltpu.VMEM((CH,), jnp.float32),
            pltpu.VMEM((4 * CH,), jnp.int32),
            pltpu.VMEM((4 * CH,), jnp.float32),
            pltpu.VMEM((4 * CH,), jnp.float32),
            pltpu.VMEM((16,), jnp.float32),
            pltpu.VMEM((16,), jnp.float32),
            pltpu.VMEM_SHARED((PLANE,), jnp.float32),
            pltpu.VMEM_SHARED((PLANE,), jnp.float32),
        ],
    )
    def k(ev_hbm, pol_hbm, ts_hbm, tref_hbm, inv_hbm, zeros_hbm,
          out_w, out_t,
          ev_v, pol_v, ts_v, idx_v, w_v, wt_v, tref_v, inv_v, acc_w, acc_t):
        c = lax.axis_index("c")
        s = lax.axis_index("s")

        pltpu.sync_copy(tref_hbm, tref_v)
        pltpu.sync_copy(inv_hbm, inv_v)
        tref = tref_v[...]
        inv = inv_v[...]
        lanes = lax.iota(jnp.int32, 16)
        lanes2 = lanes * 2

        def do_groups(ngroups, nvalid):
            def group(g, _):
                rows2 = g * 32 + lanes2
                y = plsc.load_gather(ev_v, [rows2])
                x = plsc.load_gather(ev_v, [rows2 + 1])
                p = plsc.load_gather(pol_v, [rows2])
                t = ts_v[pl.ds(g * 16, 16)]
                iy = y.astype(jnp.int32)
                ix = x.astype(jnp.int32)
                fy = y - iy.astype(jnp.float32)
                fx = x - ix.astype(jnp.float32)
                nt = 1.0 - jnp.abs(tref - t) * inv
                chan = 1 - p.astype(jnp.int32)
                valid = ((iy >= 0) & (iy <= H - 2) & (ix >= 0) & (ix <= W - 2))
                base = jnp.where(valid, chan * HW + iy * W + ix, 0)
                wy0 = 1.0 - fy
                wx0 = 1.0 - fx
                w00 = jnp.where(valid, wy0 * wx0, 0.0)
                w01 = jnp.where(valid, wy0 * fx, 0.0)
                w10 = jnp.where(valid, fy * wx0, 0.0)
                w11 = jnp.where(valid, fy * fx, 0.0)
                o = g * 64
                idx_v[pl.ds(o, 16)] = base
                idx_v[pl.ds(o + 16, 16)] = base + 1
                idx_v[pl.ds(o + 32, 16)] = base + W
                idx_v[pl.ds(o + 48, 16)] = base + W + 1
                w_v[pl.ds(o, 16)] = w00
                w_v[pl.ds(o + 16, 16)] = w01
                w_v[pl.ds(o + 32, 16)] = w10
                w_v[pl.ds(o + 48, 16)] = w11
                wt_v[pl.ds(o, 16)] = w00 * nt
                wt_v[pl.ds(o + 16, 16)] = w01 * nt
                wt_v[pl.ds(o + 32, 16)] = w10 * nt
                wt_v[pl.ds(o + 48, 16)] = w11 * nt
                return 0

            lax.fori_loop(0, ngroups, group, 0)
            zerov = tref * 0.0
            zeroi = lanes * 0

            def zgroup(g, _):
                o = g * 16
                w_v[pl.ds(o, 16)] = zerov
                wt_v[pl.ds(o, 16)] = zerov
                idx_v[pl.ds(o, 16)] = zeroi
                return 0

            if nvalid is not None:
                lax.fori_loop(nvalid * 4, 4 * G, zgroup, 0)

        for bi in range(BPC):
            b = c * BPC + bi
            pltpu.sync_copy(zeros_hbm, acc_w.at[pl.ds(s * STRIPE, STRIPE)])
            pltpu.sync_copy(zeros_hbm, acc_t.at[pl.ds(s * STRIPE, STRIPE)])
            plsc.subcore_barrier()

            def chunk(j, _):
                off = s * PER_TILE + j * CH
                pltpu.sync_copy(ev_hbm.at[pl.ds((b * N + off) * 2, 2 * CH)], ev_v)
                pltpu.sync_copy(pol_hbm.at[pl.ds((b * 4 * N + off) * 2, 2 * CH)], pol_v)
                pltpu.sync_copy(ts_hbm.at[pl.ds(b * N + off, CH)], ts_v)
                do_groups(G, None)
                pltpu.sync_copy(w_v, acc_w.at[idx_v], add=True)
                pltpu.sync_copy(wt_v, acc_t.at[idx_v], add=True)
                return 0

            lax.fori_loop(0, NCHUNK, chunk, 0)

            if tail > 0:
                @pl.when(s == NS - 1)
                def _():
                    pltpu.sync_copy(ev_hbm.at[pl.ds((b * N + covered) * 2, 2 * tail)],
                                    ev_v.at[pl.ds(0, 2 * tail)])
                    pltpu.sync_copy(pol_hbm.at[pl.ds((b * 4 * N + covered) * 2, 2 * tail)],
                                    pol_v.at[pl.ds(0, 2 * tail)])
                    pltpu.sync_copy(ts_hbm.at[pl.ds(b * N + covered, tail)],
                                    ts_v.at[pl.ds(0, tail)])
                    do_groups(TGROUPS, TGROUPS)
                    pltpu.sync_copy(w_v, acc_w.at[idx_v], add=True)
                    pltpu.sync_copy(wt_v, acc_t.at[idx_v], add=True)

            plsc.subcore_barrier()
            pltpu.sync_copy(acc_w.at[pl.ds(s * STRIPE, STRIPE)],
                            out_w.at[pl.ds(b * PLANE + s * STRIPE, STRIPE)])
            pltpu.sync_copy(acc_t.at[pl.ds(s * STRIPE, STRIPE)],
                            out_t.at[pl.ds(b * PLANE + s * STRIPE, STRIPE)])

    return k


def kernel(warped_events, pol_mask, ts_list, tref, ts_scaling):
    B, N, _ = warped_events.shape
    k = _make_sc_kernel(B, N)
    ev = warped_events.reshape(B * N * 2)
    pol = pol_mask.reshape(B * 4 * N * 2)
    ts = ts_list.reshape(B * N)
    tref16 = jnp.full((16,), tref[0], dtype=jnp.float32)
    inv16 = jnp.full((16,), 1.0 / ts_scaling[0], dtype=jnp.float32)
    zeros = jnp.zeros((STRIPE,), dtype=jnp.float32)
    out_w, out_t = k(ev, pol, ts, tref16, inv16, zeros)
    return (out_w.reshape(B, 2, H, W), out_t.reshape(B, 2, H, W))
